# Initial kernel scaffold; baseline (speedup 1.0000x reference)
import functools

import jax
import jax.numpy as jnp
from jax import lax
from jax.experimental import pallas as pl
from jax.experimental.pallas import tpu as pltpu
from jax.experimental.pallas import tpu_sc as plsc


@functools.cache
def _make_gather(V, D, B):
    info = plsc.get_sparse_core_info()
    NC, NS = info.num_cores, info.num_subcores
    NW = NC * NS
    assert B % NW == 0
    b_per_w = B // NW
    chunk = 1024
    while b_per_w % chunk:
        chunk //= 2
    n_chunks = b_per_w // chunk
    mesh = plsc.VectorSubcoreMesh(core_axis_name="c", subcore_axis_name="s")

    @functools.partial(
        pl.kernel,
        out_type=jax.ShapeDtypeStruct((B, D), jnp.float32),
        mesh=mesh,
        scratch_types=[
            pltpu.VMEM((chunk,), jnp.int32),
            pltpu.VMEM((chunk, D), jnp.float32),
            pltpu.SemaphoreType.DMA,
        ],
    )
    def gather_kernel(table_hbm, idx_hbm, out_hbm, idx_v, rows_v, sem):
        wid = lax.axis_index("s") * NC + lax.axis_index("c")
        base = wid * b_per_w

        def body(c, carry):
            off = base + c * chunk
            pltpu.sync_copy(idx_hbm.at[pl.ds(off, chunk)], idx_v)
            pltpu.async_copy(table_hbm.at[idx_v], rows_v, sem).wait()
            pltpu.sync_copy(rows_v, out_hbm.at[pl.ds(off, chunk)])
            return carry

        lax.fori_loop(0, n_chunks, body, 0)

    return gather_kernel


def kernel(embed_id, weight):
    bsz, hist = embed_id.shape
    V, D = weight.shape
    B = bsz * hist
    flat_idx = embed_id.reshape(B).astype(jnp.int32)
    out = _make_gather(V, D, B)(weight, flat_idx)
    return out.reshape(bsz, hist, D)


# speedup vs baseline: 1.4595x; 1.4595x over previous
import functools

import jax
import jax.numpy as jnp
from jax import lax
from jax.experimental import pallas as pl
from jax.experimental.pallas import tpu as pltpu
from jax.experimental.pallas import tpu_sc as plsc


@functools.cache
def _make_gather(V, D, B):
    info = plsc.get_sparse_core_info()
    NC, NS = info.num_cores, info.num_subcores
    NW = NC * NS
    assert B % NW == 0
    b_per_w = B // NW
    chunk = 1024
    while b_per_w % chunk:
        chunk //= 2
    n_chunks = b_per_w // chunk
    mesh = plsc.VectorSubcoreMesh(core_axis_name="c", subcore_axis_name="s")

    @functools.partial(
        pl.kernel,
        out_type=jax.ShapeDtypeStruct((B, D), jnp.float32),
        mesh=mesh,
        scratch_types=[
            pltpu.VMEM((chunk,), jnp.int32),
            pltpu.VMEM((chunk, D), jnp.float32),
            pltpu.SemaphoreType.DMA,
        ],
        compiler_params=pltpu.CompilerParams(use_tc_tiling_on_sc=False),
    )
    def gather_kernel(table_hbm, idx_hbm, out_hbm, idx_v, rows_v, sem):
        wid = lax.axis_index("s") * NC + lax.axis_index("c")
        base = wid * b_per_w

        def body(c, carry):
            off = base + c * chunk
            pltpu.sync_copy(idx_hbm.at[pl.ds(off, chunk)], idx_v)
            pltpu.async_copy(table_hbm.at[idx_v], rows_v, sem).wait()
            pltpu.sync_copy(rows_v, out_hbm.at[pl.ds(off, chunk)])
            return carry

        lax.fori_loop(0, n_chunks, body, 0)

    return gather_kernel


def kernel(embed_id, weight):
    bsz, hist = embed_id.shape
    V, D = weight.shape
    B = bsz * hist
    flat_idx = embed_id.reshape(B).astype(jnp.int32)
    out = _make_gather(V, D, B)(weight, flat_idx)
    return out.reshape(bsz, hist, D)


# speedup vs baseline: 1.4992x; 1.0271x over previous
import functools

import jax
import jax.numpy as jnp
from jax import lax
from jax.experimental import pallas as pl
from jax.experimental.pallas import tpu as pltpu
from jax.experimental.pallas import tpu_sc as plsc


@functools.cache
def _make_gather(V, D, B):
    info = plsc.get_sparse_core_info()
    NC, NS = info.num_cores, info.num_subcores
    NW = NC * NS
    assert B % NW == 0
    b_per_w = B // NW
    chunk = 512
    while b_per_w % chunk:
        chunk //= 2
    n_chunks = b_per_w // chunk
    nbuf = min(4, n_chunks)
    total = -(-(n_chunks + 1) // nbuf) * nbuf
    mesh = plsc.VectorSubcoreMesh(core_axis_name="c", subcore_axis_name="s")

    @functools.partial(
        pl.kernel,
        out_type=jax.ShapeDtypeStruct((B, D), jnp.float32),
        mesh=mesh,
        scratch_types=[
            pltpu.VMEM((b_per_w,), jnp.int32),
            [pltpu.VMEM((chunk, D), jnp.float32) for _ in range(nbuf)],
            [pltpu.SemaphoreType.DMA for _ in range(nbuf)],
            [pltpu.SemaphoreType.DMA for _ in range(nbuf)],
        ],
        compiler_params=pltpu.CompilerParams(use_tc_tiling_on_sc=False),
    )
    def gather_kernel(table_hbm, idx_hbm, out_hbm, idx_v, rows_v, sem_g, sem_w):
        wid = lax.axis_index("s") * NC + lax.axis_index("c")
        base = wid * b_per_w
        pltpu.sync_copy(idx_hbm.at[pl.ds(base, b_per_w)], idx_v)

        def issue_gather(c, b):
            pltpu.async_copy(
                table_hbm.at[idx_v.at[pl.ds(c * chunk, chunk)]],
                rows_v[b],
                sem_g[b],
            )

        def issue_wb(c, b):
            pltpu.async_copy(
                rows_v[b],
                out_hbm.at[pl.ds(base + c * chunk, chunk)],
                sem_w[b],
            )

        def wait_g(b):
            pltpu.make_async_copy(
                out_hbm.at[pl.ds(0, chunk)], rows_v[b], sem_g[b]
            ).wait()

        def wait_w(b):
            pltpu.make_async_copy(
                rows_v[b], out_hbm.at[pl.ds(0, chunk)], sem_w[b]
            ).wait()

        def outer(cc):
            for j in range(nbuf):
                c = cc + j

                @pl.when(c < n_chunks)
                def _issue():
                    @pl.when(c >= nbuf)
                    def _free():
                        wait_w(j)

                    issue_gather(c, j)

                d = c - 1
                jd = (j - 1) % nbuf

                @pl.when((d >= 0) & (d < n_chunks))
                def _drain():
                    wait_g(jd)
                    issue_wb(d, jd)

        pl.loop(0, total, step=nbuf)(outer)

        for j in range(nbuf):
            wait_w(j)

    return gather_kernel


def kernel(embed_id, weight):
    bsz, hist = embed_id.shape
    V, D = weight.shape
    B = bsz * hist
    flat_idx = embed_id.reshape(B).astype(jnp.int32)
    out = _make_gather(V, D, B)(weight, flat_idx)
    return out.reshape(bsz, hist, D)


# speedup vs baseline: 1.5762x; 1.0514x over previous
import functools

import jax
import jax.numpy as jnp
from jax import lax
from jax.experimental import pallas as pl
from jax.experimental.pallas import tpu as pltpu
from jax.experimental.pallas import tpu_sc as plsc


@functools.cache
def _make_gather(V, D, B):
    info = plsc.get_sparse_core_info()
    NC, NS = info.num_cores, info.num_subcores
    NW = NC * NS
    assert B % NW == 0
    b_per_w = B // NW
    chunk = 512
    while b_per_w % chunk:
        chunk //= 2
    n_chunks = b_per_w // chunk
    nbuf = min(4, n_chunks)
    total = -(-(n_chunks + 1) // nbuf) * nbuf
    mesh = plsc.VectorSubcoreMesh(core_axis_name="c", subcore_axis_name="s")

    @functools.partial(
        pl.kernel,
        out_type=jax.ShapeDtypeStruct((B, D), jnp.float32),
        mesh=mesh,
        scratch_types=[
            pltpu.VMEM((b_per_w,), jnp.int32),
            [pltpu.VMEM((chunk, D), jnp.float32) for _ in range(nbuf)],
            [pltpu.SemaphoreType.DMA for _ in range(nbuf)],
            [pltpu.SemaphoreType.DMA for _ in range(nbuf)],
        ],
        compiler_params=pltpu.CompilerParams(use_tc_tiling_on_sc=False),
    )
    def gather_kernel(table_hbm, idx_hbm, out_hbm, idx_v, rows_v, sem_g, sem_w):
        wid = lax.axis_index("s") * NC + lax.axis_index("c")
        base = wid * b_per_w
        pltpu.sync_copy(idx_hbm.at[pl.ds(base, b_per_w)], idx_v)

        def issue_gather(c, b):
            pltpu.async_copy(
                table_hbm.at[idx_v.at[pl.ds(c * chunk, chunk)]],
                rows_v[b],
                sem_g[b],
            )

        def issue_wb(c, b):
            pltpu.async_copy(
                rows_v[b],
                out_hbm.at[pl.ds(base + c * chunk, chunk)],
                sem_w[b],
            )

        def wait_g(b):
            pltpu.make_async_copy(
                out_hbm.at[pl.ds(0, chunk)], rows_v[b], sem_g[b]
            ).wait()

        def wait_w(b):
            pltpu.make_async_copy(
                rows_v[b], out_hbm.at[pl.ds(0, chunk)], sem_w[b]
            ).wait()

        def outer(cc):
            for j in range(nbuf):
                c = cc + j

                @pl.when(c < n_chunks)
                def _issue():
                    @pl.when(c >= nbuf)
                    def _free():
                        wait_w(j)

                    issue_gather(c, j)

                d = c - 1
                jd = (j - 1) % nbuf

                @pl.when((d >= 0) & (d < n_chunks))
                def _drain():
                    wait_g(jd)
                    issue_wb(d, jd)

        pl.loop(0, total, step=nbuf)(outer)

        for j in range(nbuf):
            wait_w(j)

    return gather_kernel


def kernel(embed_id, weight):
    bsz, hist = embed_id.shape
    V, D = weight.shape
    B = bsz * hist
    flat_idx = jnp.transpose(embed_id).reshape(B).astype(jnp.int32)
    out = _make_gather(V, D, B)(weight, flat_idx)
    return jnp.transpose(out.reshape(hist, bsz, D), (1, 0, 2))
